# baseline (speedup 1.0000x reference)
import functools

import jax
import jax.numpy as jnp
from jax import lax
from jax.experimental import pallas as pl
from jax.experimental.pallas import tpu as pltpu
from jax.experimental.pallas import tpu_sc as plsc

_B = 16384
_EMB = 64
_NB = 1024
_CH = 256
_NS = 8
_PD = 5
_CAP = 4


def _make_sc_gather():
    info = plsc.get_sparse_core_info()
    nc, ns = info.num_cores, info.num_subcores
    nw = nc * ns
    bpw = _B // nw
    mesh = plsc.VectorSubcoreMesh(core_axis_name="c", subcore_axis_name="s")

    @functools.partial(
        pl.kernel,
        mesh=mesh,
        compiler_params=pltpu.CompilerParams(needs_layout_passes=False),
        out_type=[
            jax.ShapeDtypeStruct((_B, _EMB), jnp.float32),
            jax.ShapeDtypeStruct((_B, _EMB), jnp.float32),
        ],
        scratch_types=[
            pltpu.VMEM((bpw,), jnp.int32),
            pltpu.VMEM((bpw,), jnp.int32),
            pltpu.VMEM((bpw,), jnp.int32),
            pltpu.VMEM((bpw,), jnp.int32),
        ] + [pltpu.VMEM((_EMB, 128), jnp.float32)] * _NS + [
            pltpu.VMEM((_CH, _EMB), jnp.float32),
        ] + [pltpu.SemaphoreType.DMA] * (_NS + 1),
    )
    def gather(su_hbm, si_hbm, pu_hbm, pi_hbm, utT_hbm, itT_hbm,
               ue_hbm, ie_hbm, uidx, iidx, uprm, iprm, *rest):
        blks = rest[:_NS]
        xout = rest[_NS]
        sems = rest[_NS + 1:2 * _NS + 1]
        osem = rest[2 * _NS + 1]
        wid = lax.axis_index("s") * nc + lax.axis_index("c")
        base = wid * bpw
        pltpu.sync_copy(su_hbm.at[pl.ds(base, bpw)], uidx)
        pltpu.sync_copy(si_hbm.at[pl.ds(base, bpw)], iidx)
        pltpu.sync_copy(pu_hbm.at[pl.ds(base, bpw)], uprm)
        pltpu.sync_copy(pi_hbm.at[pl.ds(base, bpw)], iprm)
        kvecs = [lax.iota(jnp.int32, 16) + 16 * m for m in range(_EMB // 16)]

        def fire(tbl, e, slot):
            c0 = pl.multiple_of(lax.bitwise_and(e, jnp.int32(-128)), 128)
            pltpu.async_copy(tbl.at[:, pl.ds(c0, 128)], blks[slot], sems[slot])

        def chain(pos, prevb, e):
            bk = lax.shift_right_logical(e, 7)
            same = jnp.logical_and(bk == prevb, pos < _CAP - 1)
            return lax.select(same, pos + 1, jnp.int32(0)), bk

        def extract(tbl, e, k, j_local, pos):
            @pl.when(pos == 0)
            def _():
                pltpu.make_async_copy(tbl.at[:, pl.ds(0, 128)],
                                      blks[k % _NS], sems[k % _NS]).wait()
            l = lax.bitwise_and(e, jnp.int32(127))
            lvec = jnp.full((16,), l, jnp.int32)
            jvec = jnp.full((16,), j_local, jnp.int32)
            for r in range(_CAP):
                @pl.when(pos == r)
                def _(r=r):
                    s = (k - r) % _NS
                    for m in range(_EMB // 16):
                        x = plsc.load_gather(blks[s], [kvecs[m], lvec])
                        plsc.store_scatter(xout, [jvec, kvecs[m]], x)

        def run_table(tbl, idx, prm, out_hbm):
            def chunk_body(ch, cc):
                off = ch * _CH
                v0 = idx[pl.ds(off, 16)]
                pos_f = jnp.int32(0)
                prevb_f = lax.shift_right_logical(v0[0], 7)
                fire(tbl, v0[0], 0)
                for k in range(1, _PD):
                    pos_f, prevb_f = chain(pos_f, prevb_f, v0[k])

                    @pl.when(pos_f == 0)
                    def _(k=k):
                        fire(tbl, v0[k], k % _NS)

                def step(g, carry, v, vn, tail):
                    pos_f, prevb_f, pos_e, prevb_e = carry
                    for k in range(16):
                        j = g * 16 + k
                        e = v[k]
                        pos_e, prevb_e = chain(pos_e, prevb_e, e)
                        extract(tbl, e, k, j, pos_e)
                        if not tail or k + _PD < 16:
                            nk = k + _PD
                            en = v[nk] if nk < 16 else vn[nk - 16]
                            pos_f, prevb_f = chain(pos_f, prevb_f, en)
                            slot = nk % _NS

                            @pl.when(pos_f == 0)
                            def _(en=en, slot=slot):
                                fire(tbl, en, slot)
                    return pos_f, prevb_f, pos_e, prevb_e

                def group(g, carry):
                    v = idx[pl.ds(off + g * 16, 16)]
                    vn = idx[pl.ds(off + g * 16 + 16, 16)]
                    return step(g, carry, v, vn, False)

                carry = (pos_f, prevb_f, jnp.int32(0), jnp.int32(-1))
                carry = lax.fori_loop(0, _CH // 16 - 1, group, carry)
                gl = _CH // 16 - 1
                vl = idx[pl.ds(off + gl * 16, 16)]
                step(jnp.int32(gl), carry, vl, vl, True)

                def scat(q, c):
                    pv = prm[pl.ds(off + q * 16, 16)]
                    for k in range(16):
                        pltpu.async_copy(
                            xout.at[pl.ds(q * 16 + k, 1)],
                            out_hbm.at[pl.ds(pv[k], 1)], osem)
                    return c
                lax.fori_loop(0, _CH // 16, scat, 0)
                pltpu.make_async_copy(xout, out_hbm.at[pl.ds(0, _CH)],
                                      osem).wait()
                return cc

            lax.fori_loop(0, bpw // _CH, chunk_body, 0)

        run_table(utT_hbm, uidx, uprm, ue_hbm)
        run_table(itT_hbm, iidx, iprm, ie_hbm)

    return gather


_sc_gather = None


def _mlp_body(ue_ref, ie_ref, w0a_ref, w0b_ref, b0_ref, w1_ref, b1_ref,
              w2_ref, b2_ref, w3_ref, b3_ref, out_ref):
    h = ue_ref[...] @ w0a_ref[...] + ie_ref[...] @ w0b_ref[...] + b0_ref[...]
    h = jnp.maximum(h, 0.0)
    h = jnp.maximum(h @ w1_ref[...] + b1_ref[...], 0.0)
    h = jnp.maximum(h @ w2_ref[...] + b2_ref[...], 0.0)
    out_ref[...] = (h @ w3_ref[...] + b3_ref[...])[:, 0]


def _tc_mlp(ue, ie, w0a, w0b, b0, w1t, b1, w2t, b2, w3t, b3):
    grid = (_B // _NB,)
    full = lambda shape: pl.BlockSpec(shape, lambda i: (0,) * len(shape))
    return pl.pallas_call(
        _mlp_body,
        grid=grid,
        in_specs=[
            pl.BlockSpec((_NB, _EMB), lambda i: (i, 0)),
            pl.BlockSpec((_NB, _EMB), lambda i: (i, 0)),
            full(w0a.shape), full(w0b.shape), full(b0.shape),
            full(w1t.shape), full(b1.shape),
            full(w2t.shape), full(b2.shape),
            full(w3t.shape), full(b3.shape),
        ],
        out_specs=pl.BlockSpec((_NB,), lambda i: (i,)),
        out_shape=jax.ShapeDtypeStruct((_B,), jnp.float32),
    )(ue, ie, w0a, w0b, b0, w1t, b1, w2t, b2, w3t, b3)


def kernel(users, items, user_table, item_table, W0, b0, W1, b1, W2, b2, W3, b3):
    global _sc_gather
    if _sc_gather is None:
        _sc_gather = _make_sc_gather()
    users = users.astype(jnp.int32)
    items = items.astype(jnp.int32)
    iota = lax.iota(jnp.int32, _B)
    su, pu = lax.sort_key_val(users, iota)
    si, pi = lax.sort_key_val(items, iota)
    ue, ie = _sc_gather(su, si, pu, pi, user_table.T, item_table.T)
    w0t = W0.T
    w0a, w0b = w0t[:_EMB], w0t[_EMB:]
    return _tc_mlp(
        ue, ie,
        w0a, w0b, b0.reshape(1, -1),
        W1.T, b1.reshape(1, -1),
        W2.T, b2.reshape(1, -1),
        W3.T, b3.reshape(1, -1),
    )


# speedup vs baseline: 1.0328x; 1.0328x over previous
import functools

import jax
import jax.numpy as jnp
from jax import lax
from jax.experimental import pallas as pl
from jax.experimental.pallas import tpu as pltpu
from jax.experimental.pallas import tpu_sc as plsc

_B = 16384
_EMB = 64
_NB = 1024
_CH = 256
_NS = 8
_PD = 6
_CAP = 3


def _make_sc_gather():
    info = plsc.get_sparse_core_info()
    nc, ns = info.num_cores, info.num_subcores
    nw = nc * ns
    bpw = _B // nw
    mesh = plsc.VectorSubcoreMesh(core_axis_name="c", subcore_axis_name="s")

    @functools.partial(
        pl.kernel,
        mesh=mesh,
        compiler_params=pltpu.CompilerParams(needs_layout_passes=False),
        out_type=[
            jax.ShapeDtypeStruct((_B, _EMB), jnp.float32),
            jax.ShapeDtypeStruct((_B, _EMB), jnp.float32),
        ],
        scratch_types=[
            pltpu.VMEM((bpw,), jnp.int32),
            pltpu.VMEM((bpw,), jnp.int32),
            pltpu.VMEM((bpw,), jnp.int32),
            pltpu.VMEM((bpw,), jnp.int32),
        ] + [pltpu.VMEM((_EMB, 128), jnp.float32)] * _NS + [
            pltpu.VMEM((_CH, _EMB), jnp.float32),
        ] + [pltpu.SemaphoreType.DMA] * (_NS + 1),
    )
    def gather(su_hbm, si_hbm, pu_hbm, pi_hbm, utT_hbm, itT_hbm,
               ue_hbm, ie_hbm, uidx, iidx, uprm, iprm, *rest):
        blks = rest[:_NS]
        xout = rest[_NS]
        sems = rest[_NS + 1:2 * _NS + 1]
        osem = rest[2 * _NS + 1]
        wid = lax.axis_index("s") * nc + lax.axis_index("c")
        base = wid * bpw
        pltpu.sync_copy(su_hbm.at[pl.ds(base, bpw)], uidx)
        pltpu.sync_copy(si_hbm.at[pl.ds(base, bpw)], iidx)
        pltpu.sync_copy(pu_hbm.at[pl.ds(base, bpw)], uprm)
        pltpu.sync_copy(pi_hbm.at[pl.ds(base, bpw)], iprm)
        kvecs = [lax.iota(jnp.int32, 16) + 16 * m for m in range(_EMB // 16)]

        def fire(tbl, e, slot):
            c0 = pl.multiple_of(lax.bitwise_and(e, jnp.int32(-128)), 128)
            pltpu.async_copy(tbl.at[:, pl.ds(c0, 128)], blks[slot], sems[slot])

        def chain(pos, prevb, e):
            bk = lax.shift_right_logical(e, 7)
            same = jnp.logical_and(bk == prevb, pos < _CAP - 1)
            return lax.select(same, pos + 1, jnp.int32(0)), bk

        def extract(tbl, e, k, j_local, pos):
            @pl.when(pos == 0)
            def _():
                pltpu.make_async_copy(tbl.at[:, pl.ds(0, 128)],
                                      blks[k % _NS], sems[k % _NS]).wait()
            l = lax.bitwise_and(e, jnp.int32(127))
            lvec = jnp.full((16,), l, jnp.int32)
            jvec = jnp.full((16,), j_local, jnp.int32)
            for r in range(_CAP):
                @pl.when(pos == r)
                def _(r=r):
                    s = (k - r) % _NS
                    for m in range(_EMB // 16):
                        x = plsc.load_gather(blks[s], [kvecs[m], lvec])
                        plsc.store_scatter(xout, [jvec, kvecs[m]], x)

        def run_table(tbl, idx, prm, out_hbm):
            def chunk_body(ch, cc):
                off = ch * _CH
                v0 = idx[pl.ds(off, 16)]
                pos_f = jnp.int32(0)
                prevb_f = lax.shift_right_logical(v0[0], 7)
                fire(tbl, v0[0], 0)
                for k in range(1, _PD):
                    pos_f, prevb_f = chain(pos_f, prevb_f, v0[k])

                    @pl.when(pos_f == 0)
                    def _(k=k):
                        fire(tbl, v0[k], k % _NS)

                def step(g, carry, v, vn, tail):
                    pos_f, prevb_f, pos_e, prevb_e = carry
                    for k in range(16):
                        j = g * 16 + k
                        e = v[k]
                        pos_e, prevb_e = chain(pos_e, prevb_e, e)
                        extract(tbl, e, k, j, pos_e)
                        if not tail or k + _PD < 16:
                            nk = k + _PD
                            en = v[nk] if nk < 16 else vn[nk - 16]
                            pos_f, prevb_f = chain(pos_f, prevb_f, en)
                            slot = nk % _NS

                            @pl.when(pos_f == 0)
                            def _(en=en, slot=slot):
                                fire(tbl, en, slot)
                    return pos_f, prevb_f, pos_e, prevb_e

                def group(g, carry):
                    v = idx[pl.ds(off + g * 16, 16)]
                    vn = idx[pl.ds(off + g * 16 + 16, 16)]
                    return step(g, carry, v, vn, False)

                carry = (pos_f, prevb_f, jnp.int32(0), jnp.int32(-1))
                carry = lax.fori_loop(0, _CH // 16 - 1, group, carry)
                gl = _CH // 16 - 1
                vl = idx[pl.ds(off + gl * 16, 16)]
                step(jnp.int32(gl), carry, vl, vl, True)

                def scat(q, c):
                    pv = prm[pl.ds(off + q * 16, 16)]
                    for k in range(16):
                        pltpu.async_copy(
                            xout.at[pl.ds(q * 16 + k, 1)],
                            out_hbm.at[pl.ds(pv[k], 1)], osem)
                    return c
                lax.fori_loop(0, _CH // 16, scat, 0)
                pltpu.make_async_copy(xout, out_hbm.at[pl.ds(0, _CH)],
                                      osem).wait()
                return cc

            lax.fori_loop(0, bpw // _CH, chunk_body, 0)

        run_table(utT_hbm, uidx, uprm, ue_hbm)
        run_table(itT_hbm, iidx, iprm, ie_hbm)

    return gather


_sc_gather = None


def _mlp_body(ue_ref, ie_ref, w0a_ref, w0b_ref, b0_ref, w1_ref, b1_ref,
              w2_ref, b2_ref, w3_ref, b3_ref, out_ref):
    h = ue_ref[...] @ w0a_ref[...] + ie_ref[...] @ w0b_ref[...] + b0_ref[...]
    h = jnp.maximum(h, 0.0)
    h = jnp.maximum(h @ w1_ref[...] + b1_ref[...], 0.0)
    h = jnp.maximum(h @ w2_ref[...] + b2_ref[...], 0.0)
    out_ref[...] = (h @ w3_ref[...] + b3_ref[...])[:, 0]


def _tc_mlp(ue, ie, w0a, w0b, b0, w1t, b1, w2t, b2, w3t, b3):
    grid = (_B // _NB,)
    full = lambda shape: pl.BlockSpec(shape, lambda i: (0,) * len(shape))
    return pl.pallas_call(
        _mlp_body,
        grid=grid,
        in_specs=[
            pl.BlockSpec((_NB, _EMB), lambda i: (i, 0)),
            pl.BlockSpec((_NB, _EMB), lambda i: (i, 0)),
            full(w0a.shape), full(w0b.shape), full(b0.shape),
            full(w1t.shape), full(b1.shape),
            full(w2t.shape), full(b2.shape),
            full(w3t.shape), full(b3.shape),
        ],
        out_specs=pl.BlockSpec((_NB,), lambda i: (i,)),
        out_shape=jax.ShapeDtypeStruct((_B,), jnp.float32),
    )(ue, ie, w0a, w0b, b0, w1t, b1, w2t, b2, w3t, b3)


def kernel(users, items, user_table, item_table, W0, b0, W1, b1, W2, b2, W3, b3):
    global _sc_gather
    if _sc_gather is None:
        _sc_gather = _make_sc_gather()
    users = users.astype(jnp.int32)
    items = items.astype(jnp.int32)
    iota = lax.iota(jnp.int32, _B)
    su, pu = lax.sort_key_val(users, iota)
    si, pi = lax.sort_key_val(items, iota)
    ue, ie = _sc_gather(su, si, pu, pi, user_table.T, item_table.T)
    w0t = W0.T
    w0a, w0b = w0t[:_EMB], w0t[_EMB:]
    return _tc_mlp(
        ue, ie,
        w0a, w0b, b0.reshape(1, -1),
        W1.T, b1.reshape(1, -1),
        W2.T, b2.reshape(1, -1),
        W3.T, b3.reshape(1, -1),
    )
